# baseline (speedup 1.0000x reference)
import functools

import jax
import jax.numpy as jnp
from jax import lax
from jax.experimental import pallas as pl
from jax.experimental.pallas import tpu as pltpu
from jax.experimental.pallas import tpu_sc as plsc

NC = 2
NS = 16
NW = NC * NS
CH = 128
NB = 4
GRP = 8


def _make_gather(B, V, D):
  n_chunks = B // (NW * CH)
  assert B % (NW * CH) == 0 and n_chunks % GRP == 0

  mesh = plsc.VectorSubcoreMesh(
      core_axis_name="c", subcore_axis_name="s", num_cores=NC, num_subcores=NS
  )

  @functools.partial(
      pl.kernel,
      mesh=mesh,
      out_type=jax.ShapeDtypeStruct((B, D), jnp.float32),
      scratch_types=[
          pltpu.VMEM((n_chunks, CH), jnp.int32),
          pltpu.VMEM((NB, CH, D), jnp.float32),
          pltpu.VMEM_SHARED((NS, 2, CH, D), jnp.float32),
          pltpu.SemaphoreType.DMA((NB,)),
          pltpu.SemaphoreType.DMA((NB,)),
          pltpu.SemaphoreType.DMA((2,)),
          pltpu.SemaphoreType.DMA,
      ],
  )
  def gather(table_hbm, idx_hbm, out_hbm, idx_v, vbufs, all_sbufs, gsems,
             xsems, osems, dsem):
    cid = lax.axis_index("c")
    sid = lax.axis_index("s")
    wid = sid * NC + cid
    base = wid * (n_chunks * CH)
    sbufs = all_sbufs.at[sid]

    pltpu.sync_copy(idx_hbm.at[wid], idx_v)

    def g_copy(j, b):
      return pltpu.make_async_copy(
          table_hbm.at[idx_v.at[j]], vbufs.at[b], gsems.at[b]
      )

    def x_copy(b, sb):
      return pltpu.make_async_copy(vbufs.at[b], sbufs.at[sb], xsems.at[b])

    def o_copy(j, sb):
      return pltpu.make_async_copy(
          sbufs.at[sb], out_hbm.at[pl.ds(base + j * CH, CH)], osems.at[sb]
      )

    def d_copy(j, b):
      return pltpu.make_async_copy(
          vbufs.at[b], out_hbm.at[pl.ds(base + j * CH, CH)], dsem
      )

    def step(j, p, b):
      if p == 1:
        @pl.when(j >= GRP + 1)
        def _():
          d_copy(0, (b + 2) % NB).wait()

      @pl.when(j + 2 < n_chunks)
      def _():
        g_copy(j + 2, (b + 2) % NB).start()

      g_copy(0, b).wait()

      if p == 7:
        d_copy(j, b).start()
      else:
        sb = p % 2
        guard = 5 if p == 1 else 2

        @pl.when(j >= guard)
        def _():
          o_copy(0, sb).wait()

        x_copy(b, sb).start()

      if p != 0:
        @pl.when(j >= 1)
        def _():
          x_copy((b - 1) % NB, (p - 1) % 2).wait()
          o_copy(j - 1, (p - 1) % 2).start()

    g_copy(0, 0).start()
    g_copy(1, 1).start()

    def body(g, _):
      for p in range(GRP):
        j = GRP * g + p
        step(j, p, (GRP * g + p) % NB)
      return 0

    lax.fori_loop(0, n_chunks // GRP, body, 0)

    o_copy(0, 1).wait()
    o_copy(0, 0).wait()
    d_copy(0, 0).wait()

  return gather


def kernel(x, table):
  B0, B1 = x.shape
  V, D = table.shape
  B = B0 * B1
  idx = x.reshape(NW, B // (NW * CH), CH).astype(jnp.int32)
  out = _make_gather(B, V, D)(table, idx)
  return out.reshape(B0, B1, D)


# speedup vs baseline: 1.0042x; 1.0042x over previous
import functools

import jax
import jax.numpy as jnp
from jax import lax
from jax.experimental import pallas as pl
from jax.experimental.pallas import tpu as pltpu
from jax.experimental.pallas import tpu_sc as plsc

NC = 2
NS = 16
NW = NC * NS
CH = 128
NB = 4


def _make_gather(B, V, D):
  n_chunks = B // (NW * CH)
  assert B % (NW * CH) == 0 and n_chunks % NB == 0 and n_chunks >= 8

  mesh = plsc.VectorSubcoreMesh(
      core_axis_name="c", subcore_axis_name="s", num_cores=NC, num_subcores=NS
  )

  @functools.partial(
      pl.kernel,
      mesh=mesh,
      out_type=jax.ShapeDtypeStruct((B, D), jnp.float32),
      scratch_types=[
          pltpu.VMEM((n_chunks, CH), jnp.int32),
          pltpu.VMEM((NB, CH, D), jnp.float32),
          pltpu.VMEM_SHARED((NS, 2, CH, D), jnp.float32),
          pltpu.SemaphoreType.DMA((NB,)),
          pltpu.SemaphoreType.DMA((NB,)),
          pltpu.SemaphoreType.DMA((NB,)),
      ],
  )
  def gather(table_hbm, idx_hbm, out_hbm, idx_v, vbufs, all_sbufs, gsems,
             xsems, osems):
    cid = lax.axis_index("c")
    sid = lax.axis_index("s")
    wid = sid * NC + cid
    base = wid * (n_chunks * CH)
    sbufs = all_sbufs.at[sid]

    pltpu.sync_copy(idx_hbm.at[wid], idx_v)

    def g_copy(j, b):
      return pltpu.make_async_copy(
          table_hbm.at[idx_v.at[j]], vbufs.at[b], gsems.at[b]
      )

    def x_copy(b):
      return pltpu.make_async_copy(vbufs.at[b], sbufs.at[b % 2], xsems.at[b])

    def o_copy(j, b):
      return pltpu.make_async_copy(
          sbufs.at[b % 2], out_hbm.at[pl.ds(base + j * CH, CH)],
          osems.at[b % 2]
      )

    def step(j, b):
      @pl.when(j + 2 < n_chunks)
      def _():
        g_copy(j + 2, (b + 2) % NB).start()

      g_copy(0, b).wait()

      @pl.when(j >= 2)
      def _():
        o_copy(0, b).wait()

      x_copy(b).start()

      @pl.when(j >= 1)
      def _():
        x_copy((b - 1) % NB).wait()
        o_copy(j - 1, (b - 1) % NB).start()

    g_copy(0, 0).start()
    g_copy(1, 1).start()

    def body(g, _):
      for b in range(NB):
        step(NB * g + b, b)
      return 0

    lax.fori_loop(0, n_chunks // NB, body, 0)

    last = (n_chunks - 1) % NB
    x_copy(last).wait()
    o_copy(n_chunks - 1, last).start()
    for k in range(2):
      o_copy(0, (n_chunks - 1 - k) % NB).wait()

  return gather


def kernel(x, table):
  B0, B1 = x.shape
  V, D = table.shape
  B = B0 * B1
  idx = x.reshape(NW, B // (NW * CH), CH).astype(jnp.int32)
  out = _make_gather(B, V, D)(table, idx)
  return out.reshape(B0, B1, D)


# speedup vs baseline: 1.0161x; 1.0119x over previous
import functools

import jax
import jax.numpy as jnp
from jax import lax
from jax.experimental import pallas as pl
from jax.experimental.pallas import tpu as pltpu
from jax.experimental.pallas import tpu_sc as plsc

NC = 2
NS = 16
NW = NC * NS
CH = 128
NB = 4


def _make_gather(B, V, D):
  n_chunks = B // (NW * CH)
  assert B % (NW * CH) == 0 and n_chunks % NB == 0 and n_chunks >= 8

  mesh = plsc.VectorSubcoreMesh(
      core_axis_name="c", subcore_axis_name="s", num_cores=NC, num_subcores=NS
  )

  @functools.partial(
      pl.kernel,
      mesh=mesh,
      out_type=jax.ShapeDtypeStruct((B, D), jnp.float32),
      scratch_types=[
          pltpu.VMEM((n_chunks, CH), jnp.int32),
          pltpu.VMEM((NB, CH, D), jnp.float32),
          pltpu.VMEM_SHARED((NS, 2, CH, D), jnp.float32),
          pltpu.SemaphoreType.DMA((NB,)),
          pltpu.SemaphoreType.DMA((NB,)),
          pltpu.SemaphoreType.DMA((NB,)),
      ],
  )
  def gather(table_hbm, idx_hbm, out_hbm, idx_v, vbufs, all_sbufs, gsems,
             xsems, osems):
    cid = lax.axis_index("c")
    sid = lax.axis_index("s")
    wid = sid * NC + cid
    base = wid * (n_chunks * CH)
    sbufs = all_sbufs.at[sid]

    pltpu.sync_copy(idx_hbm.at[wid], idx_v)

    def g_copy(j, b):
      return pltpu.make_async_copy(
          table_hbm.at[idx_v.at[j]], vbufs.at[b], gsems.at[b]
      )

    def x_copy(b):
      return pltpu.make_async_copy(vbufs.at[b], sbufs.at[b % 2], xsems.at[b])

    def o_copy(j, b):
      return pltpu.make_async_copy(
          sbufs.at[b % 2], out_hbm.at[pl.ds(base + j * CH, CH)],
          osems.at[b % 2]
      )

    def step(j, b):
      @pl.when(j + 2 < n_chunks)
      def _():
        g_copy(j + 2, (b + 2) % NB).start()

      @pl.when(j >= 1)
      def _():
        x_copy((b - 1) % NB).wait()
        o_copy(j - 1, (b - 1) % NB).start()

      g_copy(0, b).wait()

      @pl.when(j >= 2)
      def _():
        o_copy(0, b).wait()

      x_copy(b).start()

    g_copy(0, 0).start()
    g_copy(1, 1).start()

    def body(g, _):
      for b in range(NB):
        step(NB * g + b, b)
      return 0

    lax.fori_loop(0, n_chunks // NB, body, 0)

    last = (n_chunks - 1) % NB
    x_copy(last).wait()
    o_copy(n_chunks - 1, last).start()
    for k in range(2):
      o_copy(0, (n_chunks - 1 - k) % NB).wait()

  return gather


def kernel(x, table):
  B0, B1 = x.shape
  V, D = table.shape
  B = B0 * B1
  idx = x.reshape(NW, B // (NW * CH), CH).astype(jnp.int32)
  out = _make_gather(B, V, D)(table, idx)
  return out.reshape(B0, B1, D)
